# Initial kernel scaffold; baseline (speedup 1.0000x reference)
import jax
import jax.numpy as jnp
from jax.experimental import pallas as pl


def kernel(x, edge_index, W1, b1, W2, b2):
    raise NotImplementedError("write your pallas kernel here")


# speedup vs baseline: 23.5640x; 23.5640x over previous
import functools

import jax
import jax.numpy as jnp
from jax import lax
from jax.experimental import pallas as pl
from jax.experimental.pallas import tpu as pltpu
from jax.experimental.pallas import tpu_sc as plsc

N = 10000
E = 320000
D = 128
HID = 32
NCLS = 16

NC, NS, L = 2, 16, 16
NW = NC * NS
SB = 128
NPAD = 10240
EPAD = 327680
NSUB = EPAD // (NW * SB)
RPT = NPAD // NS
BLK = 512
GRID = NPAD // BLK


def _fill(buf, rows, width, val):
    def body(i, carry):
        for c in range(width // L):
            buf[i, pl.ds(c * L, L)] = jnp.full((L,), val, jnp.float32)
        return carry
    lax.fori_loop(0, rows, body, 0)


def _sc_mesh():
    return plsc.VectorSubcoreMesh(core_axis_name="c", subcore_axis_name="s")


_SC_PARAMS = pltpu.CompilerParams(use_tc_tiling_on_sc=False)


@functools.partial(
    pl.kernel,
    out_type=jax.ShapeDtypeStruct((NC, NPAD, L), jnp.float32),
    mesh=_sc_mesh(),
    compiler_params=_SC_PARAMS,
    scratch_types=[
        pltpu.VMEM((NSUB, SB), jnp.int32),
        pltpu.VMEM((SB, L), jnp.float32),
        pltpu.VMEM_SHARED((NPAD, L), jnp.float32),
    ],
)
def _deg_kernel(row2d, degp, rowv, onesb, acc):
    cid = lax.axis_index("c")
    sid = lax.axis_index("s")
    wid = sid * NC + cid
    pltpu.sync_copy(row2d.at[pl.ds(wid * NSUB, NSUB)], rowv)
    _fill(onesb, SB, L, 0.0)
    for k in range(RPT // SB):
        pltpu.sync_copy(onesb, acc.at[pl.ds(sid * RPT + k * SB, SB)])
    _fill(onesb, SB, L, 1.0)
    plsc.subcore_barrier()

    def body(j, carry):
        pltpu.sync_copy(onesb, acc.at[rowv.at[j]], add=True)
        return carry
    lax.fori_loop(0, NSUB, body, 0)
    plsc.subcore_barrier()
    pltpu.sync_copy(acc.at[pl.ds(sid * RPT, RPT)],
                    degp.at[cid, pl.ds(sid * RPT, RPT)])


def _make_gs(width):
    @functools.partial(
        pl.kernel,
        out_type=jax.ShapeDtypeStruct((NC, NPAD, width), jnp.float32),
        mesh=_sc_mesh(),
        compiler_params=_SC_PARAMS,
        scratch_types=[
            pltpu.VMEM((NSUB, SB), jnp.int32),
            pltpu.VMEM((NSUB, SB), jnp.int32),
            pltpu.VMEM((SB, width), jnp.float32),
            pltpu.VMEM_SHARED((NPAD, width), jnp.float32),
        ],
    )
    def gs(col2d, row2d, y, accp, colv, rowv, gbuf, acc):
        cid = lax.axis_index("c")
        sid = lax.axis_index("s")
        wid = sid * NC + cid
        pltpu.sync_copy(col2d.at[pl.ds(wid * NSUB, NSUB)], colv)
        pltpu.sync_copy(row2d.at[pl.ds(wid * NSUB, NSUB)], rowv)
        _fill(gbuf, SB, width, 0.0)
        for k in range(RPT // SB):
            pltpu.sync_copy(gbuf, acc.at[pl.ds(sid * RPT + k * SB, SB)])
        plsc.subcore_barrier()

        def body(j, carry):
            pltpu.sync_copy(y.at[colv.at[j]], gbuf)
            pltpu.sync_copy(gbuf, acc.at[rowv.at[j]], add=True)
            return carry
        lax.fori_loop(0, NSUB, body, 0)
        plsc.subcore_barrier()
        pltpu.sync_copy(acc.at[pl.ds(sid * RPT, RPT)],
                        accp.at[cid, pl.ds(sid * RPT, RPT)])
    return gs


_gs32 = _make_gs(HID)
_gs16 = _make_gs(NCLS)


def _dinv_from(degv):
    deg = degv[0, :, 0:1] + degv[1, :, 0:1]
    return jnp.where(deg > 0, lax.rsqrt(jnp.maximum(deg, 1.0)), 0.0)


def _stage1_body(x_ref, deg_ref, b1_ref, y1_ref):
    dinv = _dinv_from(deg_ref[...])
    y1_ref[...] = dinv * jnp.dot(x_ref[...], b1_ref[...],
                                 preferred_element_type=jnp.float32)


def _stage1(x_pad, degp, B1):
    return pl.pallas_call(
        _stage1_body,
        grid=(GRID,),
        in_specs=[
            pl.BlockSpec((BLK, D), lambda i: (i, 0)),
            pl.BlockSpec((NC, BLK, L), lambda i: (0, i, 0)),
            pl.BlockSpec((D, HID), lambda i: (0, 0)),
        ],
        out_specs=pl.BlockSpec((BLK, HID), lambda i: (i, 0)),
        out_shape=jax.ShapeDtypeStruct((NPAD, HID), jnp.float32),
    )(x_pad, degp, B1)


def _stage2_body(x_ref, deg_ref, s1_ref, a1_ref, b2_ref, c1_ref,
                 h_ref, y2_ref):
    dinv = _dinv_from(deg_ref[...])
    s1 = s1_ref[0] + s1_ref[1]
    h = jnp.dot(x_ref[...], a1_ref[...], preferred_element_type=jnp.float32)
    h = jnp.maximum(h - dinv * s1 + c1_ref[0:1, :], 0.0)
    h_ref[...] = h
    y2_ref[...] = dinv * jnp.dot(h, b2_ref[...],
                                 preferred_element_type=jnp.float32)


def _stage2(x_pad, degp, s1p, A1, B2, c1p):
    return pl.pallas_call(
        _stage2_body,
        grid=(GRID,),
        in_specs=[
            pl.BlockSpec((BLK, D), lambda i: (i, 0)),
            pl.BlockSpec((NC, BLK, L), lambda i: (0, i, 0)),
            pl.BlockSpec((NC, BLK, HID), lambda i: (0, i, 0)),
            pl.BlockSpec((D, HID), lambda i: (0, 0)),
            pl.BlockSpec((HID, NCLS), lambda i: (0, 0)),
            pl.BlockSpec((8, HID), lambda i: (0, 0)),
        ],
        out_specs=[
            pl.BlockSpec((BLK, HID), lambda i: (i, 0)),
            pl.BlockSpec((BLK, NCLS), lambda i: (i, 0)),
        ],
        out_shape=[
            jax.ShapeDtypeStruct((NPAD, HID), jnp.float32),
            jax.ShapeDtypeStruct((NPAD, NCLS), jnp.float32),
        ],
    )(x_pad, degp, s1p, A1, B2, c1p)


def _stage3_body(h_ref, deg_ref, s2_ref, a2_ref, c2_ref, out_ref):
    dinv = _dinv_from(deg_ref[...])
    s2 = s2_ref[0] + s2_ref[1]
    z = jnp.dot(h_ref[...], a2_ref[...], preferred_element_type=jnp.float32)
    z = z - dinv * s2 + c2_ref[0:1, :]
    z = z - jnp.max(z, axis=1, keepdims=True)
    out_ref[...] = z - jnp.log(jnp.sum(jnp.exp(z), axis=1, keepdims=True))


def _stage3(h, degp, s2p, A2, c2p):
    return pl.pallas_call(
        _stage3_body,
        grid=(GRID,),
        in_specs=[
            pl.BlockSpec((BLK, HID), lambda i: (i, 0)),
            pl.BlockSpec((NC, BLK, L), lambda i: (0, i, 0)),
            pl.BlockSpec((NC, BLK, NCLS), lambda i: (0, i, 0)),
            pl.BlockSpec((HID, NCLS), lambda i: (0, 0)),
            pl.BlockSpec((8, NCLS), lambda i: (0, 0)),
        ],
        out_specs=pl.BlockSpec((BLK, NCLS), lambda i: (i, 0)),
        out_shape=jax.ShapeDtypeStruct((N, NCLS), jnp.float32),
    )(h, degp, s2p, A2, c2p)


def kernel(x, edge_index, W1, b1, W2, b2):
    ei = edge_index.astype(jnp.int32)
    pad_idx = N + (jnp.arange(EPAD - E, dtype=jnp.int32) % 8)
    row2d = jnp.concatenate([ei[0], pad_idx]).reshape(EPAD // SB, SB)
    col2d = jnp.concatenate([ei[1], pad_idx]).reshape(EPAD // SB, SB)
    x_pad = jnp.concatenate(
        [x, jnp.zeros((NPAD - N, D), jnp.float32)], axis=0)

    A1 = W1[:, 0].sum(0)
    B1 = W1[:, 1].sum(0)
    A2 = W2[:, 0].sum(0)
    B2 = W2[:, 1].sum(0)
    c1p = jnp.zeros((8, HID), jnp.float32).at[0].set(b1.sum(0))
    c2p = jnp.zeros((8, NCLS), jnp.float32).at[0].set(b2.sum(0))

    degp = _deg_kernel(row2d)
    y1 = _stage1(x_pad, degp, B1)
    s1p = _gs32(col2d, row2d, y1)
    h, y2 = _stage2(x_pad, degp, s1p, A1, B2, c1p)
    s2p = _gs16(col2d, row2d, y2)
    return _stage3(h, degp, s2p, A2, c2p)


# speedup vs baseline: 43.3434x; 1.8394x over previous
import functools

import jax
import jax.numpy as jnp
from jax import lax
from jax.experimental import pallas as pl
from jax.experimental.pallas import tpu as pltpu
from jax.experimental.pallas import tpu_sc as plsc

N = 10000
E = 320000
D = 128
HID = 32
NCLS = 16

NC, NS, L = 2, 16, 16
NW = NC * NS
SB = 128
NPAD = 10240
EPAD = 327680
NSUB = EPAD // (NW * SB)
RPT = NPAD // NS
BLK = 512
GRID = NPAD // BLK


def _fill(buf, rows, width, val):
    def body(i, carry):
        for c in range(width // L):
            buf[i, pl.ds(c * L, L)] = jnp.full((L,), val, jnp.float32)
        return carry
    lax.fori_loop(0, rows, body, 0)


def _sc_mesh():
    return plsc.VectorSubcoreMesh(core_axis_name="c", subcore_axis_name="s")


_SC_PARAMS = pltpu.CompilerParams(use_tc_tiling_on_sc=False)


@functools.partial(
    pl.kernel,
    out_type=jax.ShapeDtypeStruct((NC, NPAD, L), jnp.float32),
    mesh=_sc_mesh(),
    compiler_params=_SC_PARAMS,
    scratch_types=[
        pltpu.VMEM((NSUB, SB), jnp.int32),
        pltpu.VMEM((SB, L), jnp.float32),
        pltpu.VMEM_SHARED((NPAD, L), jnp.float32),
        pltpu.SemaphoreType.DMA,
    ],
)
def _deg_kernel(row2d, degp, rowv, onesb, acc, ssem):
    cid = lax.axis_index("c")
    sid = lax.axis_index("s")
    wid = sid * NC + cid
    pltpu.sync_copy(row2d.at[pl.ds(wid * NSUB, NSUB)], rowv)
    _fill(onesb, SB, L, 0.0)
    for k in range(RPT // SB):
        pltpu.sync_copy(onesb, acc.at[pl.ds(sid * RPT + k * SB, SB)])
    _fill(onesb, SB, L, 1.0)
    plsc.subcore_barrier()

    def body(j, carry):
        pltpu.async_copy(onesb, acc.at[rowv.at[j]], ssem, add=True)
        return carry
    lax.fori_loop(0, NSUB, body, 0)

    def drain(j, carry):
        pltpu.make_async_copy(onesb, acc.at[rowv.at[0]], ssem).wait()
        return carry
    lax.fori_loop(0, NSUB, drain, 0)
    plsc.subcore_barrier()
    pltpu.sync_copy(acc.at[pl.ds(sid * RPT, RPT)],
                    degp.at[cid, pl.ds(sid * RPT, RPT)])


GK = 8
NGRP = NSUB // GK


def _make_gs(width):
    @functools.partial(
        pl.kernel,
        out_type=jax.ShapeDtypeStruct((NC, NPAD, width), jnp.float32),
        mesh=_sc_mesh(),
        compiler_params=_SC_PARAMS,
        scratch_types=[
            pltpu.VMEM((NSUB, SB), jnp.int32),
            pltpu.VMEM((NSUB, SB), jnp.int32),
            pltpu.VMEM((GK, SB, width), jnp.float32),
            pltpu.VMEM((GK, SB, width), jnp.float32),
            pltpu.VMEM_SHARED((NPAD, width), jnp.float32),
            pltpu.SemaphoreType.DMA,
            pltpu.SemaphoreType.DMA,
            pltpu.SemaphoreType.DMA,
            pltpu.SemaphoreType.DMA,
        ],
    )
    def gs(col2d, row2d, y, accp, colv, rowv, bufa, bufb, acc,
           gsema, gsemb, ssema, ssemb):
        cid = lax.axis_index("c")
        sid = lax.axis_index("s")
        wid = sid * NC + cid
        pltpu.sync_copy(col2d.at[pl.ds(wid * NSUB, NSUB)], colv)
        pltpu.sync_copy(row2d.at[pl.ds(wid * NSUB, NSUB)], rowv)
        _fill(bufa.at[0], SB, width, 0.0)
        for k in range(RPT // SB):
            pltpu.sync_copy(bufa.at[0], acc.at[pl.ds(sid * RPT + k * SB, SB)])
        plsc.subcore_barrier()

        def fire_gather(base, buf, sem):
            for b in range(GK):
                pltpu.async_copy(y.at[colv.at[base + b]], buf.at[b], sem)

        def drain_gather(buf, sem):
            for b in range(GK):
                pltpu.make_async_copy(y.at[colv.at[0]], buf.at[b], sem).wait()

        def scatter(base, buf, sem):
            descs = [
                pltpu.async_copy(buf.at[b], acc.at[rowv.at[base + b]], sem,
                                 add=True)
                for b in range(GK)
            ]
            for d in descs:
                d.wait()

        fire_gather(0, bufa, gsema)

        def body(t, carry):
            base = 2 * t * GK
            drain_gather(bufa, gsema)
            fire_gather(base + GK, bufb, gsemb)
            scatter(base, bufa, ssema)

            @pl.when(t < NGRP // 2 - 1)
            def _():
                fire_gather(base + 2 * GK, bufa, gsema)
            drain_gather(bufb, gsemb)
            scatter(base + GK, bufb, ssemb)
            return carry
        lax.fori_loop(0, NGRP // 2, body, 0)
        plsc.subcore_barrier()
        pltpu.sync_copy(acc.at[pl.ds(sid * RPT, RPT)],
                        accp.at[cid, pl.ds(sid * RPT, RPT)])
    return gs


_gs32 = _make_gs(HID)
_gs16 = _make_gs(NCLS)


def _dinv_from(degv):
    deg = degv[0, :, 0:1] + degv[1, :, 0:1]
    return jnp.where(deg > 0, lax.rsqrt(jnp.maximum(deg, 1.0)), 0.0)


def _stage1_body(x_ref, deg_ref, b1_ref, y1_ref):
    dinv = _dinv_from(deg_ref[...])
    y1_ref[...] = dinv * jnp.dot(x_ref[...], b1_ref[...],
                                 preferred_element_type=jnp.float32)


def _stage1(x_pad, degp, B1):
    return pl.pallas_call(
        _stage1_body,
        grid=(GRID,),
        in_specs=[
            pl.BlockSpec((BLK, D), lambda i: (i, 0)),
            pl.BlockSpec((NC, BLK, L), lambda i: (0, i, 0)),
            pl.BlockSpec((D, HID), lambda i: (0, 0)),
        ],
        out_specs=pl.BlockSpec((BLK, HID), lambda i: (i, 0)),
        out_shape=jax.ShapeDtypeStruct((NPAD, HID), jnp.float32),
    )(x_pad, degp, B1)


def _stage2_body(x_ref, deg_ref, s1_ref, a1_ref, b2_ref, c1_ref,
                 h_ref, y2_ref):
    dinv = _dinv_from(deg_ref[...])
    s1 = s1_ref[0] + s1_ref[1]
    h = jnp.dot(x_ref[...], a1_ref[...], preferred_element_type=jnp.float32)
    h = jnp.maximum(h - dinv * s1 + c1_ref[0:1, :], 0.0)
    h_ref[...] = h
    y2_ref[...] = dinv * jnp.dot(h, b2_ref[...],
                                 preferred_element_type=jnp.float32)


def _stage2(x_pad, degp, s1p, A1, B2, c1p):
    return pl.pallas_call(
        _stage2_body,
        grid=(GRID,),
        in_specs=[
            pl.BlockSpec((BLK, D), lambda i: (i, 0)),
            pl.BlockSpec((NC, BLK, L), lambda i: (0, i, 0)),
            pl.BlockSpec((NC, BLK, HID), lambda i: (0, i, 0)),
            pl.BlockSpec((D, HID), lambda i: (0, 0)),
            pl.BlockSpec((HID, NCLS), lambda i: (0, 0)),
            pl.BlockSpec((8, HID), lambda i: (0, 0)),
        ],
        out_specs=[
            pl.BlockSpec((BLK, HID), lambda i: (i, 0)),
            pl.BlockSpec((BLK, NCLS), lambda i: (i, 0)),
        ],
        out_shape=[
            jax.ShapeDtypeStruct((NPAD, HID), jnp.float32),
            jax.ShapeDtypeStruct((NPAD, NCLS), jnp.float32),
        ],
    )(x_pad, degp, s1p, A1, B2, c1p)


def _stage3_body(h_ref, deg_ref, s2_ref, a2_ref, c2_ref, out_ref):
    dinv = _dinv_from(deg_ref[...])
    s2 = s2_ref[0] + s2_ref[1]
    z = jnp.dot(h_ref[...], a2_ref[...], preferred_element_type=jnp.float32)
    z = z - dinv * s2 + c2_ref[0:1, :]
    z = z - jnp.max(z, axis=1, keepdims=True)
    out_ref[...] = z - jnp.log(jnp.sum(jnp.exp(z), axis=1, keepdims=True))


def _stage3(h, degp, s2p, A2, c2p):
    return pl.pallas_call(
        _stage3_body,
        grid=(GRID,),
        in_specs=[
            pl.BlockSpec((BLK, HID), lambda i: (i, 0)),
            pl.BlockSpec((NC, BLK, L), lambda i: (0, i, 0)),
            pl.BlockSpec((NC, BLK, NCLS), lambda i: (0, i, 0)),
            pl.BlockSpec((HID, NCLS), lambda i: (0, 0)),
            pl.BlockSpec((8, NCLS), lambda i: (0, 0)),
        ],
        out_specs=pl.BlockSpec((BLK, NCLS), lambda i: (i, 0)),
        out_shape=jax.ShapeDtypeStruct((N, NCLS), jnp.float32),
    )(h, degp, s2p, A2, c2p)


def kernel(x, edge_index, W1, b1, W2, b2):
    ei = edge_index.astype(jnp.int32)
    pad_idx = N + (jnp.arange(EPAD - E, dtype=jnp.int32) % (NPAD - N))
    row2d = jnp.concatenate([ei[0], pad_idx]).reshape(EPAD // SB, SB)
    col2d = jnp.concatenate([ei[1], pad_idx]).reshape(EPAD // SB, SB)
    x_pad = jnp.concatenate(
        [x, jnp.zeros((NPAD - N, D), jnp.float32)], axis=0)

    A1 = W1[:, 0].sum(0)
    B1 = W1[:, 1].sum(0)
    A2 = W2[:, 0].sum(0)
    B2 = W2[:, 1].sum(0)
    c1p = jnp.zeros((8, HID), jnp.float32).at[0].set(b1.sum(0))
    c2p = jnp.zeros((8, NCLS), jnp.float32).at[0].set(b2.sum(0))

    degp = _deg_kernel(row2d)
    y1 = _stage1(x_pad, degp, B1)
    s1p = _gs32(col2d, row2d, y1)
    h, y2 = _stage2(x_pad, degp, s1p, A1, B2, c1p)
    s2p = _gs16(col2d, row2d, y2)
    return _stage3(h, degp, s2p, A2, c2p)


# speedup vs baseline: 50.7557x; 1.1710x over previous
import functools

import jax
import jax.numpy as jnp
from jax import lax
from jax.experimental import pallas as pl
from jax.experimental.pallas import tpu as pltpu
from jax.experimental.pallas import tpu_sc as plsc

N = 10000
E = 320000
D = 128
HID = 32
NCLS = 16

NC, NS, L = 2, 16, 16
NW = NC * NS
SB = 128
NPAD = 10240
EPAD = 327680
NSUB = EPAD // (NW * SB)
RPT = NPAD // NS
BLK = 2048
GRID = NPAD // BLK


def _fill(buf, rows, width, val):
    def body(i, carry):
        for c in range(width // L):
            buf[i, pl.ds(c * L, L)] = jnp.full((L,), val, jnp.float32)
        return carry
    lax.fori_loop(0, rows, body, 0)


def _sc_mesh():
    return plsc.VectorSubcoreMesh(core_axis_name="c", subcore_axis_name="s")


_SC_PARAMS = pltpu.CompilerParams(use_tc_tiling_on_sc=False)


@functools.partial(
    pl.kernel,
    out_type=jax.ShapeDtypeStruct((NC, NPAD, L), jnp.float32),
    mesh=_sc_mesh(),
    compiler_params=_SC_PARAMS,
    scratch_types=[
        pltpu.VMEM((NSUB, SB), jnp.int32),
        pltpu.VMEM((SB, L), jnp.float32),
        pltpu.VMEM_SHARED((NPAD, L), jnp.float32),
        pltpu.SemaphoreType.DMA,
    ],
)
def _deg_kernel(e3, degp, rowv, onesb, acc, ssem):
    cid = lax.axis_index("c")
    sid = lax.axis_index("s")
    wid = sid * NC + cid
    pltpu.sync_copy(e3.at[0, pl.ds(wid * NSUB, NSUB)], rowv)
    _fill(onesb, SB, L, 0.0)
    for k in range(RPT // SB):
        pltpu.sync_copy(onesb, acc.at[pl.ds(sid * RPT + k * SB, SB)])
    _fill(onesb, SB, L, 1.0)
    plsc.subcore_barrier()

    def body(j, carry):
        pltpu.async_copy(onesb, acc.at[rowv.at[j]], ssem, add=True)
        return carry
    lax.fori_loop(0, NSUB, body, 0)

    def drain(j, carry):
        pltpu.make_async_copy(onesb, acc.at[rowv.at[0]], ssem).wait()
        return carry
    lax.fori_loop(0, NSUB, drain, 0)
    plsc.subcore_barrier()
    pltpu.sync_copy(acc.at[pl.ds(sid * RPT, RPT)],
                    degp.at[cid, pl.ds(sid * RPT, RPT)])


GK = 8
NGRP = NSUB // GK


def _make_gs(width):
    @functools.partial(
        pl.kernel,
        out_type=jax.ShapeDtypeStruct((NC, NPAD, width), jnp.float32),
        mesh=_sc_mesh(),
        compiler_params=_SC_PARAMS,
        scratch_types=[
            pltpu.VMEM((NSUB, SB), jnp.int32),
            pltpu.VMEM((NSUB, SB), jnp.int32),
            pltpu.VMEM((GK, SB, width), jnp.float32),
            pltpu.VMEM((GK, SB, width), jnp.float32),
            pltpu.VMEM_SHARED((NPAD, width), jnp.float32),
            pltpu.SemaphoreType.DMA,
            pltpu.SemaphoreType.DMA,
            pltpu.SemaphoreType.DMA,
            pltpu.SemaphoreType.DMA,
        ],
    )
    def gs(e3, y, accp, colv, rowv, bufa, bufb, acc,
           gsema, gsemb, ssema, ssemb):
        cid = lax.axis_index("c")
        sid = lax.axis_index("s")
        wid = sid * NC + cid
        pltpu.sync_copy(e3.at[1, pl.ds(wid * NSUB, NSUB)], colv)
        pltpu.sync_copy(e3.at[0, pl.ds(wid * NSUB, NSUB)], rowv)
        _fill(bufa.at[0], SB, width, 0.0)
        for k in range(RPT // SB):
            pltpu.sync_copy(bufa.at[0], acc.at[pl.ds(sid * RPT + k * SB, SB)])
        plsc.subcore_barrier()

        def fire_gather(base, buf, sem):
            for b in range(GK):
                pltpu.async_copy(y.at[colv.at[base + b]], buf.at[b], sem)

        def drain_gather(buf, sem):
            for b in range(GK):
                pltpu.make_async_copy(y.at[colv.at[0]], buf.at[b], sem).wait()

        def scatter(base, buf, sem):
            descs = [
                pltpu.async_copy(buf.at[b], acc.at[rowv.at[base + b]], sem,
                                 add=True)
                for b in range(GK)
            ]
            for d in descs:
                d.wait()

        fire_gather(0, bufa, gsema)

        def body(t, carry):
            base = 2 * t * GK
            drain_gather(bufa, gsema)
            fire_gather(base + GK, bufb, gsemb)
            scatter(base, bufa, ssema)

            @pl.when(t < NGRP // 2 - 1)
            def _():
                fire_gather(base + 2 * GK, bufa, gsema)
            drain_gather(bufb, gsemb)
            scatter(base + GK, bufb, ssemb)
            return carry
        lax.fori_loop(0, NGRP // 2, body, 0)
        plsc.subcore_barrier()
        pltpu.sync_copy(acc.at[pl.ds(sid * RPT, RPT)],
                        accp.at[cid, pl.ds(sid * RPT, RPT)])
    return gs


_gs32 = _make_gs(HID)
_gs16 = _make_gs(NCLS)


def _dinv_from(degv):
    deg = degv[0, :, 0:1] + degv[1, :, 0:1]
    return jnp.where(deg > 0, lax.rsqrt(jnp.maximum(deg, 1.0)), 0.0)


def _proj_body(x_ref, b1_ref, a1_ref, xb1_ref, xa1_ref):
    xv = x_ref[...]
    xb1_ref[...] = jnp.dot(xv, b1_ref[...], preferred_element_type=jnp.float32)
    xa1_ref[...] = jnp.dot(xv, a1_ref[...], preferred_element_type=jnp.float32)


def _t_proj(x_pad, B1, A1):
    return pl.pallas_call(
        _proj_body,
        grid=(GRID,),
        in_specs=[
            pl.BlockSpec((BLK, D), lambda i: (i, 0)),
            pl.BlockSpec((D, HID), lambda i: (0, 0)),
            pl.BlockSpec((D, HID), lambda i: (0, 0)),
        ],
        out_specs=[
            pl.BlockSpec((BLK, HID), lambda i: (i, 0)),
            pl.BlockSpec((BLK, HID), lambda i: (i, 0)),
        ],
        out_shape=[
            jax.ShapeDtypeStruct((NPAD, HID), jnp.float32),
            jax.ShapeDtypeStruct((NPAD, HID), jnp.float32),
        ],
    )(x_pad, B1, A1)


def _scale_body(deg_ref, xb1_ref, y1_ref):
    y1_ref[...] = _dinv_from(deg_ref[...]) * xb1_ref[...]


def _t_scale(degp, xb1):
    return pl.pallas_call(
        _scale_body,
        grid=(GRID,),
        in_specs=[
            pl.BlockSpec((NC, BLK, L), lambda i: (0, i, 0)),
            pl.BlockSpec((BLK, HID), lambda i: (i, 0)),
        ],
        out_specs=pl.BlockSpec((BLK, HID), lambda i: (i, 0)),
        out_shape=jax.ShapeDtypeStruct((NPAD, HID), jnp.float32),
    )(degp, xb1)


def _mid_body(deg_ref, xa1_ref, s1_ref, b2_ref, a2_ref, c1_ref, c2_ref,
              y2_ref, z0_ref):
    dinv = _dinv_from(deg_ref[...])
    s1 = s1_ref[0] + s1_ref[1]
    h = jnp.maximum(xa1_ref[...] - dinv * s1 + c1_ref[0:1, :], 0.0)
    y2_ref[...] = dinv * jnp.dot(h, b2_ref[...],
                                 preferred_element_type=jnp.float32)
    z0_ref[...] = jnp.dot(h, a2_ref[...],
                          preferred_element_type=jnp.float32) + c2_ref[0:1, :]


def _t_mid(degp, xa1, s1p, B2, A2, c1p, c2p):
    return pl.pallas_call(
        _mid_body,
        grid=(GRID,),
        in_specs=[
            pl.BlockSpec((NC, BLK, L), lambda i: (0, i, 0)),
            pl.BlockSpec((BLK, HID), lambda i: (i, 0)),
            pl.BlockSpec((NC, BLK, HID), lambda i: (0, i, 0)),
            pl.BlockSpec((HID, NCLS), lambda i: (0, 0)),
            pl.BlockSpec((HID, NCLS), lambda i: (0, 0)),
            pl.BlockSpec((8, HID), lambda i: (0, 0)),
            pl.BlockSpec((8, NCLS), lambda i: (0, 0)),
        ],
        out_specs=[
            pl.BlockSpec((BLK, NCLS), lambda i: (i, 0)),
            pl.BlockSpec((BLK, NCLS), lambda i: (i, 0)),
        ],
        out_shape=[
            jax.ShapeDtypeStruct((NPAD, NCLS), jnp.float32),
            jax.ShapeDtypeStruct((NPAD, NCLS), jnp.float32),
        ],
    )(degp, xa1, s1p, B2, A2, c1p, c2p)


def _out_body(deg_ref, z0_ref, s2_ref, out_ref):
    dinv = _dinv_from(deg_ref[...])
    z = z0_ref[...] - dinv * (s2_ref[0] + s2_ref[1])
    z = z - jnp.max(z, axis=1, keepdims=True)
    out_ref[...] = z - jnp.log(jnp.sum(jnp.exp(z), axis=1, keepdims=True))


def _t_out(degp, z0, s2p):
    return pl.pallas_call(
        _out_body,
        grid=(GRID,),
        in_specs=[
            pl.BlockSpec((NC, BLK, L), lambda i: (0, i, 0)),
            pl.BlockSpec((BLK, NCLS), lambda i: (i, 0)),
            pl.BlockSpec((NC, BLK, NCLS), lambda i: (0, i, 0)),
        ],
        out_specs=pl.BlockSpec((BLK, NCLS), lambda i: (i, 0)),
        out_shape=jax.ShapeDtypeStruct((N, NCLS), jnp.float32),
    )(degp, z0, s2p)


def kernel(x, edge_index, W1, b1, W2, b2):
    ei = edge_index.astype(jnp.int32)
    pad_idx = N + (jnp.arange(EPAD - E, dtype=jnp.int32) % (NPAD - N))
    e3 = jnp.concatenate(
        [ei, jnp.broadcast_to(pad_idx, (2, EPAD - E))], axis=1
    ).reshape(2, EPAD // SB, SB)
    x_pad = jnp.concatenate(
        [x, jnp.zeros((NPAD - N, D), jnp.float32)], axis=0)

    A1 = W1[:, 0].sum(0)
    B1 = W1[:, 1].sum(0)
    A2 = W2[:, 0].sum(0)
    B2 = W2[:, 1].sum(0)
    c1p = jnp.zeros((8, HID), jnp.float32).at[0].set(b1.sum(0))
    c2p = jnp.zeros((8, NCLS), jnp.float32).at[0].set(b2.sum(0))

    degp = _deg_kernel(e3)
    xb1, xa1 = _t_proj(x_pad, B1, A1)
    y1 = _t_scale(degp, xb1)
    s1p = _gs32(e3, y1)
    y2, z0 = _t_mid(degp, xa1, s1p, B2, A2, c1p, c2p)
    s2p = _gs16(e3, y2)
    return _t_out(degp, z0, s2p)
